# baseline (speedup 1.0000x reference)
import functools

import jax
import jax.numpy as jnp
from jax import lax
from jax.experimental import pallas as pl
from jax.experimental.pallas import tpu as pltpu
from jax.experimental.pallas import tpu_sc as plsc

_LN2 = 0.6931471805599453


def _ssp(x):
    return jnp.log(0.5 + 0.5 * jnp.exp(x))


def _pack_pairs(lo, hi):
    bl = jax.lax.bitcast_convert_type(lo, jnp.uint32)
    bh = jax.lax.bitcast_convert_type(hi, jnp.uint32)
    return jax.lax.bitcast_convert_type(
        (bl >> 16) | (bh & jnp.uint32(0xFFFF0000)), jnp.float32)


def _xi_body(x_ref, w_ref, b_ref, o0_ref, o1_ref):
    xi = jnp.dot(x_ref[...], w_ref[...], preferred_element_type=jnp.float32)
    xi = xi + b_ref[...]
    o0_ref[...] = xi[:, :128]
    o1_ref[...] = xi[:, 128:]


def _make_xi(N, H, F, bn):
    half = pl.BlockSpec((bn, F // 2), lambda i: (i, 0))
    return pl.pallas_call(
        _xi_body,
        grid=(N // bn,),
        in_specs=[
            pl.BlockSpec((bn, H), lambda i: (i, 0)),
            pl.BlockSpec((H, F), lambda i: (0, 0)),
            pl.BlockSpec((1, F), lambda i: (0, 0)),
        ],
        out_specs=[half, half],
        out_shape=[jax.ShapeDtypeStruct((N, F // 2), jnp.float32)] * 2,
    )


def _ew_body(ea_ref, w1_ref, b1_ref, w2_ref, b2_ref, o0_ref, o1_ref):
    h = jax.lax.dot_general(ea_ref[...], w1_ref[...],
                            (((0,), (0,)), ((), ())),
                            preferred_element_type=jnp.float32)
    h = _ssp(h + b1_ref[...])
    h = jnp.dot(h, w2_ref[...], preferred_element_type=jnp.float32)
    h = _ssp(h + b2_ref[...])
    o0_ref[...] = _pack_pairs(h[:, 0:64], h[:, 64:128])
    o1_ref[...] = _pack_pairs(h[:, 128:192], h[:, 192:256])


def _make_ew(Eh, G, F, be, off):
    half = pl.BlockSpec((be, F // 4), lambda i: (i, 0))
    return pl.pallas_call(
        _ew_body,
        grid=(Eh // be,),
        in_specs=[
            pl.BlockSpec((G, be), lambda i: (0, i + off)),
            pl.BlockSpec((G, F), lambda i: (0, 0)),
            pl.BlockSpec((1, F), lambda i: (0, 0)),
            pl.BlockSpec((F, F), lambda i: (0, 0)),
            pl.BlockSpec((1, F), lambda i: (0, 0)),
        ],
        out_specs=[half, half],
        out_shape=[jax.ShapeDtypeStruct((Eh, F // 4), jnp.float32)] * 2,
    )


def _make_conv(N, E, e_base):
    K = 80
    n_chunks = E // K
    NS = 16
    rounds = (n_chunks + NS - 1) // NS
    stripe = ((N + NS - 1) // NS + K - 1) // K * K
    NP = stripe * NS
    nfull = stripe // K
    last_full = N - (N % K)
    tail = N % K

    mesh = plsc.VectorSubcoreMesh(core_axis_name="c", subcore_axis_name="s")

    @functools.partial(
        pl.kernel,
        mesh=mesh,
        out_type=[jax.ShapeDtypeStruct((N, 128), jnp.float32)] * 2,
        scratch_types=[
            pltpu.VMEM((K,), jnp.int32),
            pltpu.VMEM((K,), jnp.int32),
            pltpu.VMEM((K, 128), jnp.float32),
            pltpu.VMEM((K, 64), jnp.float32),
            pltpu.VMEM((K,), jnp.int32),
            pltpu.VMEM((K,), jnp.int32),
            pltpu.VMEM((K, 128), jnp.float32),
            pltpu.VMEM((K, 64), jnp.float32),
            pltpu.VMEM_SHARED((NP, 128), jnp.float32),
            pltpu.SemaphoreType.DMA,
            pltpu.SemaphoreType.DMA,
            pltpu.SemaphoreType.DMA,
            pltpu.SemaphoreType.DMA,
        ],
    )
    def conv(xi0_hbm, xi1_hbm, row_hbm, col_hbm, ew0_hbm, ew1_hbm,
             o0_hbm, o1_hbm, colv0, rowv0, rows0, ewv0, colv1, rowv1,
             rows1, ewv1, acc, sg0, se0, sg1, se1):
        c = lax.axis_index("c")
        s = lax.axis_index("s")

        def zrow(r, _):
            for j in range(8):
                rows0[r, pl.ds(j * 16, 16)] = jnp.zeros((16,), jnp.float32)
            return ()
        lax.fori_loop(0, K, zrow, ())
        r0 = s * stripe
        for i in range(nfull):
            pltpu.sync_copy(rows0, acc.at[pl.ds(r0 + i * K, K)])
        plsc.subcore_barrier()

        bufs = ((colv0, rowv0, rows0, ewv0, sg0, se0),
                (colv1, rowv1, rows1, ewv1, sg1, se1))

        def edge_loop(xi_hbm, ew_hbm):
            def stage_a(r, buf):
                colv, rowv, rows_v, ew_v, sg, se = buf
                cidx = r * NS + s

                @pl.when(cidx < n_chunks)
                def _():
                    e0 = cidx * K
                    pltpu.sync_copy(col_hbm.at[pl.ds(e_base + e0, K)], colv)
                    pltpu.sync_copy(row_hbm.at[pl.ds(e_base + e0, K)], rowv)
                    pltpu.async_copy(xi_hbm.at[colv], rows_v, sg)
                    pltpu.async_copy(ew_hbm.at[pl.ds(e0, K)], ew_v, se)

            def stage_b(r, buf):
                colv, rowv, rows_v, ew_v, sg, se = buf
                cidx = r * NS + s

                @pl.when(cidx < n_chunks)
                def _():
                    e0 = cidx * K
                    pltpu.make_async_copy(xi_hbm.at[colv], rows_v, sg).wait()
                    pltpu.make_async_copy(
                        ew_hbm.at[pl.ds(e0, K)], ew_v, se).wait()

                    hi_mask = jnp.uint32(0xFFFF0000)
                    bc = jax.lax.bitcast_convert_type

                    def mrow(rr, _):
                        for j in range(4):
                            sl = pl.ds(j * 16, 16)
                            sh = pl.ds(64 + j * 16, 16)
                            ww = bc(ew_v[rr, sl], jnp.uint32)
                            wa = bc(ww << 16, jnp.float32)
                            wb = bc(ww & hi_mask, jnp.float32)
                            rows_v[rr, sl] = rows_v[rr, sl] * wa
                            rows_v[rr, sh] = rows_v[rr, sh] * wb
                        return ()
                    lax.fori_loop(0, K, mrow, ())
                    pltpu.sync_copy(rows_v, acc.at[rowv], add=True)

            stage_a(0, bufs[0])
            stage_a(1, bufs[1])

            def pair(h, _):
                r = h * 2
                stage_b(r, bufs[0])
                stage_a(r + 2, bufs[0])
                stage_b(r + 1, bufs[1])
                stage_a(r + 3, bufs[1])
                return ()
            lax.fori_loop(0, (rounds + 1) // 2, pair, ())

        @pl.when(c == 0)
        def _():
            edge_loop(xi0_hbm, ew0_hbm)

        @pl.when(c == 1)
        def _():
            edge_loop(xi1_hbm, ew1_hbm)

        plsc.subcore_barrier()

        def copy_out(o_hbm):
            for j in range(nfull):
                off = r0 + j * K

                @pl.when(off + K <= N)
                def _():
                    pltpu.sync_copy(acc.at[pl.ds(off, K)],
                                    o_hbm.at[pl.ds(off, K)])
            if tail:
                @pl.when((r0 <= last_full) & (last_full < r0 + stripe))
                def _():
                    pltpu.sync_copy(acc.at[pl.ds(last_full, tail)],
                                    o_hbm.at[pl.ds(last_full, tail)])

        @pl.when(c == 0)
        def _():
            copy_out(o0_hbm)

        @pl.when(c == 1)
        def _():
            copy_out(o1_hbm)

    return conv


def _epi_body(o0a_ref, o0b_ref, o1a_ref, o1b_ref, x_ref,
              wl2_ref, bl2_ref, w1_ref, b1_ref, w2_ref, b2_ref, w3_ref,
              b3_ref, out_ref):
    o = jnp.concatenate([o0a_ref[...] + o0b_ref[...],
                         o1a_ref[...] + o1b_ref[...]], axis=1)
    h = jnp.dot(o, wl2_ref[...], preferred_element_type=jnp.float32) + bl2_ref[...]
    v = _ssp(jnp.dot(h, w1_ref[...], preferred_element_type=jnp.float32) + b1_ref[...])
    v = _ssp(jnp.dot(v, w2_ref[...], preferred_element_type=jnp.float32) + b2_ref[...])
    v = jnp.dot(v, w3_ref[...], preferred_element_type=jnp.float32) + b3_ref[...]
    out_ref[...] = x_ref[...] + v


def _make_epi(N, H, bn):
    full = lambda r, c: pl.BlockSpec((r, c), lambda i: (0, 0))
    half = pl.BlockSpec((bn, H // 2), lambda i: (i, 0))
    return pl.pallas_call(
        _epi_body,
        grid=(N // bn,),
        in_specs=[
            half, half, half, half,
            pl.BlockSpec((bn, H), lambda i: (i, 0)),
            full(H, H), full(1, H),
            full(H, H), full(1, H),
            full(H, H), full(1, H),
            full(H, H), full(1, H),
        ],
        out_specs=pl.BlockSpec((bn, H), lambda i: (i, 0)),
        out_shape=jax.ShapeDtypeStruct((N, H), jnp.float32),
    )


def kernel(x, edge_index, edge_attr, W_lin1, b_lin1, W_m1, b_m1, W_m2, b_m2,
           W_lin2, b_lin2, W_u1, b_u1, W_u2, b_u2, W_u3, b_u3):
    N, H = x.shape
    E, G = edge_attr.shape
    F = W_lin1.shape[1]

    row = edge_index[0].astype(jnp.int32)
    col = edge_index[1].astype(jnp.int32)

    xi0, xi1 = _make_xi(N, H, F, 1000)(x, W_lin1, b_lin1.reshape(1, F))

    be = 640
    Ea = (E * 32 // 100) // be * be
    Eb = E - Ea
    ea_t = edge_attr.T
    ew_args = (W_m1, b_m1.reshape(1, F), W_m2, b_m2.reshape(1, F))
    ew0a, ew1a = _make_ew(Ea, G, F, be, 0)(ea_t, *ew_args)
    o0a, o1a = _make_conv(N, Ea, 0)(xi0, xi1, row, col, ew0a, ew1a)
    ew0b, ew1b = _make_ew(Eb, G, F, be, Ea // be)(ea_t, *ew_args)
    o0b, o1b = _make_conv(N, Eb, Ea)(xi0, xi1, row, col, ew0b, ew1b)
    return _make_epi(N, H, 1000)(
        o0a, o0b, o1a, o1b, x, W_lin2, b_lin2.reshape(1, H),
        W_u1, b_u1.reshape(1, H), W_u2, b_u2.reshape(1, H),
        W_u3, b_u3.reshape(1, H))


# speedup vs baseline: 1.0848x; 1.0848x over previous
import functools

import jax
import jax.numpy as jnp
from jax import lax
from jax.experimental import pallas as pl
from jax.experimental.pallas import tpu as pltpu
from jax.experimental.pallas import tpu_sc as plsc

_LN2 = 0.6931471805599453


def _ssp(x):
    return jnp.log(0.5 + 0.5 * jnp.exp(x))


def _pack_pairs(lo, hi):
    bl = jax.lax.bitcast_convert_type(lo, jnp.uint32)
    bh = jax.lax.bitcast_convert_type(hi, jnp.uint32)
    return jax.lax.bitcast_convert_type(
        (bl >> 16) | (bh & jnp.uint32(0xFFFF0000)), jnp.float32)


def _xi_body(x_ref, w_ref, b_ref, o0_ref, o1_ref):
    xi = jnp.dot(x_ref[...], w_ref[...], preferred_element_type=jnp.float32)
    xi = xi + b_ref[...]
    o0_ref[...] = xi[:, :128]
    o1_ref[...] = xi[:, 128:]


def _make_xi(N, H, F, bn):
    half = pl.BlockSpec((bn, F // 2), lambda i: (i, 0))
    return pl.pallas_call(
        _xi_body,
        grid=(N // bn,),
        in_specs=[
            pl.BlockSpec((bn, H), lambda i: (i, 0)),
            pl.BlockSpec((H, F), lambda i: (0, 0)),
            pl.BlockSpec((1, F), lambda i: (0, 0)),
        ],
        out_specs=[half, half],
        out_shape=[jax.ShapeDtypeStruct((N, F // 2), jnp.float32)] * 2,
    )


def _ew_body(ea_ref, w1_ref, b1_ref, w2_ref, b2_ref, o0_ref, o1_ref):
    h = jnp.dot(ea_ref[...], w1_ref[...], preferred_element_type=jnp.float32)
    h = _ssp(h + b1_ref[...])
    h = jnp.dot(h, w2_ref[...], preferred_element_type=jnp.float32)
    h = _ssp(h + b2_ref[...])
    o0_ref[...] = _pack_pairs(h[:, 0:64], h[:, 64:128])
    o1_ref[...] = _pack_pairs(h[:, 128:192], h[:, 192:256])


def _make_ew(Eh, G, F, be, off):
    half = pl.BlockSpec((be, F // 4), lambda i: (i, 0))
    return pl.pallas_call(
        _ew_body,
        grid=(Eh // be,),
        in_specs=[
            pl.BlockSpec((be, G), lambda i: (i + off, 0)),
            pl.BlockSpec((G, F), lambda i: (0, 0)),
            pl.BlockSpec((1, F), lambda i: (0, 0)),
            pl.BlockSpec((F, F), lambda i: (0, 0)),
            pl.BlockSpec((1, F), lambda i: (0, 0)),
        ],
        out_specs=[half, half],
        out_shape=[jax.ShapeDtypeStruct((Eh, F // 4), jnp.float32)] * 2,
    )


def _make_conv(N, E, e_base):
    K = 80
    n_chunks = E // K
    NS = 16
    rounds = (n_chunks + NS - 1) // NS
    stripe = ((N + NS - 1) // NS + K - 1) // K * K
    NP = stripe * NS
    nfull = stripe // K
    last_full = N - (N % K)
    tail = N % K

    mesh = plsc.VectorSubcoreMesh(core_axis_name="c", subcore_axis_name="s")

    @functools.partial(
        pl.kernel,
        mesh=mesh,
        out_type=[jax.ShapeDtypeStruct((N, 128), jnp.float32)] * 2,
        scratch_types=[
            pltpu.VMEM((K,), jnp.int32),
            pltpu.VMEM((K,), jnp.int32),
            pltpu.VMEM((K, 128), jnp.float32),
            pltpu.VMEM((K, 64), jnp.float32),
            pltpu.VMEM((K,), jnp.int32),
            pltpu.VMEM((K,), jnp.int32),
            pltpu.VMEM((K, 128), jnp.float32),
            pltpu.VMEM((K, 64), jnp.float32),
            pltpu.VMEM_SHARED((NP, 128), jnp.float32),
            pltpu.SemaphoreType.DMA,
            pltpu.SemaphoreType.DMA,
            pltpu.SemaphoreType.DMA,
            pltpu.SemaphoreType.DMA,
        ],
    )
    def conv(xi0_hbm, xi1_hbm, row_hbm, col_hbm, ew0_hbm, ew1_hbm,
             o0_hbm, o1_hbm, colv0, rowv0, rows0, ewv0, colv1, rowv1,
             rows1, ewv1, acc, sg0, se0, sg1, se1):
        c = lax.axis_index("c")
        s = lax.axis_index("s")

        def zrow(r, _):
            for j in range(8):
                rows0[r, pl.ds(j * 16, 16)] = jnp.zeros((16,), jnp.float32)
            return ()
        lax.fori_loop(0, K, zrow, ())
        r0 = s * stripe
        for i in range(nfull):
            pltpu.sync_copy(rows0, acc.at[pl.ds(r0 + i * K, K)])
        plsc.subcore_barrier()

        bufs = ((colv0, rowv0, rows0, ewv0, sg0, se0),
                (colv1, rowv1, rows1, ewv1, sg1, se1))

        def edge_loop(xi_hbm, ew_hbm):
            def stage_a(r, buf):
                colv, rowv, rows_v, ew_v, sg, se = buf
                cidx = r * NS + s

                @pl.when(cidx < n_chunks)
                def _():
                    e0 = cidx * K
                    pltpu.sync_copy(col_hbm.at[pl.ds(e_base + e0, K)], colv)
                    pltpu.sync_copy(row_hbm.at[pl.ds(e_base + e0, K)], rowv)
                    pltpu.async_copy(xi_hbm.at[colv], rows_v, sg)
                    pltpu.async_copy(ew_hbm.at[pl.ds(e0, K)], ew_v, se)

            def stage_b(r, buf):
                colv, rowv, rows_v, ew_v, sg, se = buf
                cidx = r * NS + s

                @pl.when(cidx < n_chunks)
                def _():
                    e0 = cidx * K
                    pltpu.make_async_copy(xi_hbm.at[colv], rows_v, sg).wait()
                    pltpu.make_async_copy(
                        ew_hbm.at[pl.ds(e0, K)], ew_v, se).wait()

                    hi_mask = jnp.uint32(0xFFFF0000)
                    bc = jax.lax.bitcast_convert_type

                    def mrow(rr, _):
                        for j in range(4):
                            sl = pl.ds(j * 16, 16)
                            sh = pl.ds(64 + j * 16, 16)
                            ww = bc(ew_v[rr, sl], jnp.uint32)
                            wa = bc(ww << 16, jnp.float32)
                            wb = bc(ww & hi_mask, jnp.float32)
                            rows_v[rr, sl] = rows_v[rr, sl] * wa
                            rows_v[rr, sh] = rows_v[rr, sh] * wb
                        return ()
                    lax.fori_loop(0, K, mrow, ())
                    pltpu.sync_copy(rows_v, acc.at[rowv], add=True)

            stage_a(0, bufs[0])
            stage_a(1, bufs[1])

            def pair(h, _):
                r = h * 2
                stage_b(r, bufs[0])
                stage_a(r + 2, bufs[0])
                stage_b(r + 1, bufs[1])
                stage_a(r + 3, bufs[1])
                return ()
            lax.fori_loop(0, (rounds + 1) // 2, pair, ())

        @pl.when(c == 0)
        def _():
            edge_loop(xi0_hbm, ew0_hbm)

        @pl.when(c == 1)
        def _():
            edge_loop(xi1_hbm, ew1_hbm)

        plsc.subcore_barrier()

        def copy_out(o_hbm):
            for j in range(nfull):
                off = r0 + j * K

                @pl.when(off + K <= N)
                def _():
                    pltpu.sync_copy(acc.at[pl.ds(off, K)],
                                    o_hbm.at[pl.ds(off, K)])
            if tail:
                @pl.when((r0 <= last_full) & (last_full < r0 + stripe))
                def _():
                    pltpu.sync_copy(acc.at[pl.ds(last_full, tail)],
                                    o_hbm.at[pl.ds(last_full, tail)])

        @pl.when(c == 0)
        def _():
            copy_out(o0_hbm)

        @pl.when(c == 1)
        def _():
            copy_out(o1_hbm)

    return conv


def _epi_body(o0a_ref, o0b_ref, o1a_ref, o1b_ref, x_ref,
              wl2_ref, bl2_ref, w1_ref, b1_ref, w2_ref, b2_ref, w3_ref,
              b3_ref, out_ref):
    o = jnp.concatenate([o0a_ref[...] + o0b_ref[...],
                         o1a_ref[...] + o1b_ref[...]], axis=1)
    h = jnp.dot(o, wl2_ref[...], preferred_element_type=jnp.float32) + bl2_ref[...]
    v = _ssp(jnp.dot(h, w1_ref[...], preferred_element_type=jnp.float32) + b1_ref[...])
    v = _ssp(jnp.dot(v, w2_ref[...], preferred_element_type=jnp.float32) + b2_ref[...])
    v = jnp.dot(v, w3_ref[...], preferred_element_type=jnp.float32) + b3_ref[...]
    out_ref[...] = x_ref[...] + v


def _make_epi(N, H, bn):
    full = lambda r, c: pl.BlockSpec((r, c), lambda i: (0, 0))
    half = pl.BlockSpec((bn, H // 2), lambda i: (i, 0))
    return pl.pallas_call(
        _epi_body,
        grid=(N // bn,),
        in_specs=[
            half, half, half, half,
            pl.BlockSpec((bn, H), lambda i: (i, 0)),
            full(H, H), full(1, H),
            full(H, H), full(1, H),
            full(H, H), full(1, H),
            full(H, H), full(1, H),
        ],
        out_specs=pl.BlockSpec((bn, H), lambda i: (i, 0)),
        out_shape=jax.ShapeDtypeStruct((N, H), jnp.float32),
    )


def kernel(x, edge_index, edge_attr, W_lin1, b_lin1, W_m1, b_m1, W_m2, b_m2,
           W_lin2, b_lin2, W_u1, b_u1, W_u2, b_u2, W_u3, b_u3):
    N, H = x.shape
    E, G = edge_attr.shape
    F = W_lin1.shape[1]

    row = edge_index[0].astype(jnp.int32)
    col = edge_index[1].astype(jnp.int32)

    xi0, xi1 = _make_xi(N, H, F, 1000)(x, W_lin1, b_lin1.reshape(1, F))

    be = 2000
    Ea = (E * 35 // 100) // be * be
    Eb = E - Ea
    ew_args = (W_m1, b_m1.reshape(1, F), W_m2, b_m2.reshape(1, F))
    ew0a, ew1a = _make_ew(Ea, G, F, be, 0)(edge_attr, *ew_args)
    o0a, o1a = _make_conv(N, Ea, 0)(xi0, xi1, row, col, ew0a, ew1a)
    ew0b, ew1b = _make_ew(Eb, G, F, be, Ea // be)(edge_attr, *ew_args)
    o0b, o1b = _make_conv(N, Eb, Ea)(xi0, xi1, row, col, ew0b, ew1b)
    return _make_epi(N, H, 1000)(
        o0a, o0b, o1a, o1b, x, W_lin2, b_lin2.reshape(1, H),
        W_u1, b_u1.reshape(1, H), W_u2, b_u2.reshape(1, H),
        W_u3, b_u3.reshape(1, H))


# speedup vs baseline: 1.1737x; 1.0819x over previous
import functools

import jax
import jax.numpy as jnp
from jax import lax
from jax.experimental import pallas as pl
from jax.experimental.pallas import tpu as pltpu
from jax.experimental.pallas import tpu_sc as plsc


def _ssp(x):
    return jnp.log(0.5 + 0.5 * jnp.exp(x))


def _pack_pairs(lo, hi):
    bl = jax.lax.bitcast_convert_type(lo, jnp.uint32)
    bh = jax.lax.bitcast_convert_type(hi, jnp.uint32)
    return jax.lax.bitcast_convert_type(
        (bl >> 16) | (bh & jnp.uint32(0xFFFF0000)), jnp.float32)


def _xi_body(x_ref, w_ref, b_ref, o0_ref, o1_ref):
    xi = jnp.dot(x_ref[...], w_ref[...], preferred_element_type=jnp.float32)
    xi = xi + b_ref[...]
    o0_ref[...] = xi[:, :128]
    o1_ref[...] = xi[:, 128:]


def _make_xi(N, H, F, bn):
    half = pl.BlockSpec((bn, F // 2), lambda i: (i, 0))
    return pl.pallas_call(
        _xi_body,
        grid=(N // bn,),
        in_specs=[
            pl.BlockSpec((bn, H), lambda i: (i, 0)),
            pl.BlockSpec((H, F), lambda i: (0, 0)),
            pl.BlockSpec((1, F), lambda i: (0, 0)),
        ],
        out_specs=[half, half],
        out_shape=[jax.ShapeDtypeStruct((N, F // 2), jnp.float32)] * 2,
    )


def _ew_body(ea_ref, w1_ref, b1_ref, w2_ref, b2_ref, o0_ref, o1_ref):
    h = jax.lax.dot_general(ea_ref[...], w1_ref[...],
                            (((0,), (0,)), ((), ())),
                            preferred_element_type=jnp.float32)
    h = _ssp(h + b1_ref[...])
    h = jnp.dot(h, w2_ref[...], preferred_element_type=jnp.float32)
    h = _ssp(h + b2_ref[...])
    o0_ref[...] = _pack_pairs(h[:, 0:64], h[:, 64:128])
    o1_ref[...] = _pack_pairs(h[:, 128:192], h[:, 192:256])


def _make_ew(Eh, G, F, be, off):
    half = pl.BlockSpec((be, F // 4), lambda i: (i, 0))
    return pl.pallas_call(
        _ew_body,
        grid=(Eh // be,),
        in_specs=[
            pl.BlockSpec((G, be), lambda i: (0, i + off)),
            pl.BlockSpec((G, F), lambda i: (0, 0)),
            pl.BlockSpec((1, F), lambda i: (0, 0)),
            pl.BlockSpec((F, F), lambda i: (0, 0)),
            pl.BlockSpec((1, F), lambda i: (0, 0)),
        ],
        out_specs=[half, half],
        out_shape=[jax.ShapeDtypeStruct((Eh, F // 4), jnp.float32)] * 2,
    )


def _make_conv(N, E, e_base):
    K = 80
    n_chunks = E // K
    NS = 16
    rounds = (n_chunks + NS - 1) // NS
    stripe = ((N + NS - 1) // NS + K - 1) // K * K
    NP = stripe * NS
    nfull = stripe // K
    last_full = N - (N % K)
    tail = N % K

    mesh = plsc.VectorSubcoreMesh(core_axis_name="c", subcore_axis_name="s")

    @functools.partial(
        pl.kernel,
        mesh=mesh,
        out_type=[jax.ShapeDtypeStruct((N, 128), jnp.float32)] * 2,
        scratch_types=[
            pltpu.VMEM((K,), jnp.int32),
            pltpu.VMEM((K,), jnp.int32),
            pltpu.VMEM((K, 128), jnp.float32),
            pltpu.VMEM((K, 64), jnp.float32),
            pltpu.VMEM((K,), jnp.int32),
            pltpu.VMEM((K,), jnp.int32),
            pltpu.VMEM((K, 128), jnp.float32),
            pltpu.VMEM((K, 64), jnp.float32),
            pltpu.VMEM_SHARED((NP, 128), jnp.float32),
            pltpu.SemaphoreType.DMA,
            pltpu.SemaphoreType.DMA,
            pltpu.SemaphoreType.DMA,
            pltpu.SemaphoreType.DMA,
        ],
    )
    def conv(xi0_hbm, xi1_hbm, row_hbm, col_hbm, ew0_hbm, ew1_hbm,
             o0_hbm, o1_hbm, colv0, rowv0, rows0, ewv0, colv1, rowv1,
             rows1, ewv1, acc, sg0, se0, sg1, se1):
        c = lax.axis_index("c")
        s = lax.axis_index("s")

        def zrow(r, _):
            for j in range(8):
                rows0[r, pl.ds(j * 16, 16)] = jnp.zeros((16,), jnp.float32)
            return ()
        lax.fori_loop(0, K, zrow, ())
        r0 = s * stripe
        for i in range(nfull):
            pltpu.sync_copy(rows0, acc.at[pl.ds(r0 + i * K, K)])
        plsc.subcore_barrier()

        bufs = ((colv0, rowv0, rows0, ewv0, sg0, se0),
                (colv1, rowv1, rows1, ewv1, sg1, se1))

        def edge_loop(xi_hbm, ew_hbm):
            def stage_a(r, buf):
                colv, rowv, rows_v, ew_v, sg, se = buf
                cidx = r * NS + s

                @pl.when(cidx < n_chunks)
                def _():
                    e0 = cidx * K
                    pltpu.sync_copy(col_hbm.at[pl.ds(e_base + e0, K)], colv)
                    pltpu.sync_copy(row_hbm.at[pl.ds(e_base + e0, K)], rowv)
                    pltpu.async_copy(xi_hbm.at[colv], rows_v, sg)
                    pltpu.async_copy(ew_hbm.at[pl.ds(e0, K)], ew_v, se)

            def stage_b(r, buf):
                colv, rowv, rows_v, ew_v, sg, se = buf
                cidx = r * NS + s

                @pl.when(cidx < n_chunks)
                def _():
                    e0 = cidx * K
                    pltpu.make_async_copy(xi_hbm.at[colv], rows_v, sg).wait()
                    pltpu.make_async_copy(
                        ew_hbm.at[pl.ds(e0, K)], ew_v, se).wait()

                    hi_mask = jnp.uint32(0xFFFF0000)
                    bc = jax.lax.bitcast_convert_type

                    def mrow(rr, _):
                        for j in range(4):
                            sl = pl.ds(j * 16, 16)
                            sh = pl.ds(64 + j * 16, 16)
                            ww = bc(ew_v[rr, sl], jnp.uint32)
                            wa = bc(ww << 16, jnp.float32)
                            wb = bc(ww & hi_mask, jnp.float32)
                            rows_v[rr, sl] = rows_v[rr, sl] * wa
                            rows_v[rr, sh] = rows_v[rr, sh] * wb
                        return ()
                    lax.fori_loop(0, K, mrow, ())
                    pltpu.sync_copy(rows_v, acc.at[rowv], add=True)

            stage_a(0, bufs[0])
            stage_a(1, bufs[1])

            def pair(h, _):
                r = h * 2
                stage_b(r, bufs[0])
                stage_a(r + 2, bufs[0])
                stage_b(r + 1, bufs[1])
                stage_a(r + 3, bufs[1])
                return ()
            lax.fori_loop(0, (rounds + 1) // 2, pair, ())

        @pl.when(c == 0)
        def _():
            edge_loop(xi0_hbm, ew0_hbm)

        @pl.when(c == 1)
        def _():
            edge_loop(xi1_hbm, ew1_hbm)

        plsc.subcore_barrier()

        def copy_out(o_hbm):
            for j in range(nfull):
                off = r0 + j * K

                @pl.when(off + K <= N)
                def _():
                    pltpu.sync_copy(acc.at[pl.ds(off, K)],
                                    o_hbm.at[pl.ds(off, K)])
            if tail:
                @pl.when((r0 <= last_full) & (last_full < r0 + stripe))
                def _():
                    pltpu.sync_copy(acc.at[pl.ds(last_full, tail)],
                                    o_hbm.at[pl.ds(last_full, tail)])

        @pl.when(c == 0)
        def _():
            copy_out(o0_hbm)

        @pl.when(c == 1)
        def _():
            copy_out(o1_hbm)

    return conv


def _epi_body(o0a_ref, o0b_ref, o1a_ref, o1b_ref, x_ref,
              wl2_ref, bl2_ref, w1_ref, b1_ref, w2_ref, b2_ref, w3_ref,
              b3_ref, out_ref):
    o = jnp.concatenate([o0a_ref[...] + o0b_ref[...],
                         o1a_ref[...] + o1b_ref[...]], axis=1)
    h = jnp.dot(o, wl2_ref[...], preferred_element_type=jnp.float32) + bl2_ref[...]
    v = _ssp(jnp.dot(h, w1_ref[...], preferred_element_type=jnp.float32) + b1_ref[...])
    v = _ssp(jnp.dot(v, w2_ref[...], preferred_element_type=jnp.float32) + b2_ref[...])
    v = jnp.dot(v, w3_ref[...], preferred_element_type=jnp.float32) + b3_ref[...]
    out_ref[...] = x_ref[...] + v


def _make_epi(N, H, bn):
    full = lambda r, c: pl.BlockSpec((r, c), lambda i: (0, 0))
    half = pl.BlockSpec((bn, H // 2), lambda i: (i, 0))
    return pl.pallas_call(
        _epi_body,
        grid=(N // bn,),
        in_specs=[
            half, half, half, half,
            pl.BlockSpec((bn, H), lambda i: (i, 0)),
            full(H, H), full(1, H),
            full(H, H), full(1, H),
            full(H, H), full(1, H),
            full(H, H), full(1, H),
        ],
        out_specs=pl.BlockSpec((bn, H), lambda i: (i, 0)),
        out_shape=jax.ShapeDtypeStruct((N, H), jnp.float32),
    )


def kernel(x, edge_index, edge_attr, W_lin1, b_lin1, W_m1, b_m1, W_m2, b_m2,
           W_lin2, b_lin2, W_u1, b_u1, W_u2, b_u2, W_u3, b_u3):
    N, H = x.shape
    E, G = edge_attr.shape
    F = W_lin1.shape[1]

    row = edge_index[0].astype(jnp.int32)
    col = edge_index[1].astype(jnp.int32)

    xi0, xi1 = _make_xi(N, H, F, 1000)(x, W_lin1, b_lin1.reshape(1, F))

    be = 3200
    Ea = (E * 32 // 100) // be * be
    Eb = E - Ea
    ea_t = edge_attr.T
    ew_args = (W_m1, b_m1.reshape(1, F), W_m2, b_m2.reshape(1, F))
    ew0a, ew1a = _make_ew(Ea, G, F, be, 0)(ea_t, *ew_args)
    o0a, o1a = _make_conv(N, Ea, 0)(xi0, xi1, row, col, ew0a, ew1a)
    ew0b, ew1b = _make_ew(Eb, G, F, be, Ea // be)(ea_t, *ew_args)
    o0b, o1b = _make_conv(N, Eb, Ea)(xi0, xi1, row, col, ew0b, ew1b)
    return _make_epi(N, H, 1000)(
        o0a, o0b, o1a, o1b, x, W_lin2, b_lin2.reshape(1, H),
        W_u1, b_u1.reshape(1, H), W_u2, b_u2.reshape(1, H),
        W_u3, b_u3.reshape(1, H))
